# baseline (speedup 1.0000x reference)
import functools

import jax
import jax.numpy as jnp
from jax import lax
from jax.experimental import pallas as pl
from jax.experimental.pallas import tpu as pltpu
from jax.experimental.pallas import tpu_sc as plsc

VOCAB = 1000000
EMB = 64
MAX_LEN = 200
CLASSES = 100
BATCH = 4096

NC = 2
NS = 16
NW = NC * NS
B_PER_W = BATCH // NW
GROUP = 2
ROWS_PER_STREAM = 100
STREAMS_PER_GROUP = GROUP * MAX_LEN // ROWS_PER_STREAM
GROUPS = B_PER_W // GROUP
IDX_ROWS_PER_W = B_PER_W * MAX_LEN // ROWS_PER_STREAM


def _pool_body(idx_hbm, table_hbm, out_hbm, idx_v, buf, stage, sem):
    wid = lax.axis_index("s") * NC + lax.axis_index("c")
    base = wid * B_PER_W

    pltpu.sync_copy(idx_hbm.at[pl.ds(wid * IDX_ROWS_PER_W, IDX_ROWS_PER_W)],
                    idx_v)

    zero = jnp.zeros((16,), jnp.float32)

    def reduce_item(item_global, base_row):
        def body(l, accs):
            r = base_row + l
            return tuple(
                accs[e] + buf[r, pl.ds(16 * e, 16)] for e in range(4)
            )
        accs = lax.fori_loop(0, MAX_LEN, body, (zero, zero, zero, zero))
        for e in range(4):
            stage[item_global, pl.ds(16 * e, 16)] = accs[e]

    def group_body(g, _):
        copies = []
        for j in range(STREAMS_PER_GROUP):
            copies.append(pltpu.async_copy(
                table_hbm.at[idx_v.at[STREAMS_PER_GROUP * g + j]],
                buf.at[pl.ds(ROWS_PER_STREAM * j, ROWS_PER_STREAM)],
                sem))
        for c in copies:
            c.wait()
        for t in range(GROUP):
            reduce_item(GROUP * g + t, MAX_LEN * t)
        return 0

    lax.fori_loop(0, GROUPS, group_body, 0)
    pltpu.sync_copy(stage, out_hbm.at[pl.ds(base, B_PER_W)])


_pool_call = functools.partial(
    pl.kernel,
    out_type=jax.ShapeDtypeStruct((BATCH, EMB), jnp.float32),
    mesh=plsc.VectorSubcoreMesh(core_axis_name="c", subcore_axis_name="s"),
    compiler_params=pltpu.CompilerParams(use_tc_tiling_on_sc=False),
    scratch_types=[
        pltpu.VMEM((IDX_ROWS_PER_W, ROWS_PER_STREAM), jnp.int32),
        pltpu.VMEM((GROUP * MAX_LEN, EMB), jnp.float32),
        pltpu.VMEM((B_PER_W, EMB), jnp.float32),
        pltpu.SemaphoreType.DMA,
    ],
)(_pool_body)


CPAD = 128
_DBLK = 512


def _dense_kernel(x_ref, w_ref, b_ref, o_ref):
    logits = jnp.dot(x_ref[...], w_ref[...],
                     preferred_element_type=jnp.float32) + b_ref[...]
    m = jnp.max(logits, axis=-1, keepdims=True)
    e = jnp.exp(logits - m)
    o_ref[...] = e / jnp.sum(e, axis=-1, keepdims=True)


_dense_call = pl.pallas_call(
    _dense_kernel,
    grid=(BATCH // _DBLK,),
    in_specs=[
        pl.BlockSpec((_DBLK, EMB), lambda i: (i, 0)),
        pl.BlockSpec((EMB, CPAD), lambda i: (0, 0)),
        pl.BlockSpec((1, CPAD), lambda i: (0, 0)),
    ],
    out_specs=pl.BlockSpec((_DBLK, CPAD), lambda i: (i, 0)),
    out_shape=jax.ShapeDtypeStruct((BATCH, CPAD), jnp.float32),
)


def kernel(inputs, table, W, b):
    idx = inputs.astype(jnp.int32).reshape(-1, ROWS_PER_STREAM)
    pool_sum = _pool_call(idx, table)
    w_pad = jnp.pad(W * (1.0 / MAX_LEN), ((0, 0), (0, CPAD - CLASSES)))
    b_pad = jnp.concatenate(
        [b, jnp.full((CPAD - CLASSES,), -1e30, b.dtype)]).reshape(1, CPAD)
    out = _dense_call(pool_sum, w_pad, b_pad)
    return out[:, :CLASSES]
